# baseline (speedup 1.0000x reference)
import jax
import jax.numpy as jnp
from jax.experimental import pallas as pl
from jax.experimental.pallas import tpu as pltpu


def _mlp_t_kernel(xt_ref, w1t_ref, b1_ref, w2_ref, b2_ref, ot_ref):
    b1t = b1_ref[...].T
    b2t = b2_ref[...].T
    h = jnp.dot(w1t_ref[...], xt_ref[...], preferred_element_type=jnp.float32)
    h = jnp.maximum(h + b1t, 0.0)
    y = jax.lax.dot_general(w2_ref[...], h, (((0,), (0,)), ((), ())),
                            preferred_element_type=jnp.float32)
    ot_ref[...] = (y + b2t).astype(ot_ref.dtype)


def _round_up(a, m):
    return ((a + m - 1) // m) * m


def kernel(x, w1, b1, w2, b2, *, block_b=32768):
    B, K = x.shape
    inter_dim = w1.shape[1]
    out_dim = w2.shape[1]

    block_b = max(128, min(block_b, _round_up(B, 128)))
    B_pad = _round_up(B, block_b)
    if B_pad != B:
        x = jnp.pad(x, ((0, B_pad - B), (0, 0)))
    num_blocks = B_pad // block_b

    xt = x.T
    w1t = w1.T

    out_t = pl.pallas_call(
        _mlp_t_kernel,
        out_shape=jax.ShapeDtypeStruct((out_dim, B_pad), x.dtype),
        grid=(num_blocks,),
        in_specs=[
            pl.BlockSpec((K, block_b), lambda i: (0, i)),
            pl.BlockSpec((inter_dim, K), lambda i: (0, 0)),
            pl.BlockSpec((1, inter_dim), lambda i: (0, 0)),
            pl.BlockSpec((inter_dim, out_dim), lambda i: (0, 0)),
            pl.BlockSpec((1, out_dim), lambda i: (0, 0)),
        ],
        out_specs=pl.BlockSpec((out_dim, block_b), lambda i: (0, i)),
        compiler_params=pltpu.CompilerParams(
            dimension_semantics=("parallel",),
            vmem_limit_bytes=64 * 1024 * 1024,
        ),
    )(xt, w1t, b1, w2, b2)

    out = out_t.T
    if B_pad != B:
        out = out[:B]
    return out


# speedup vs baseline: 1.0183x; 1.0183x over previous
import jax
import jax.numpy as jnp
from jax.experimental import pallas as pl
from jax.experimental.pallas import tpu as pltpu


def _mlp_t_kernel(xt_ref, w1t_ref, b1_ref, w2_ref, b2_ref, ot_ref):
    b1t = b1_ref[...].T
    b2t = b2_ref[...].T
    h = jnp.dot(w1t_ref[...], xt_ref[...], preferred_element_type=jnp.float32)
    h = jnp.maximum(h + b1t, 0.0)
    y = jax.lax.dot_general(w2_ref[...], h, (((0,), (0,)), ((), ())),
                            preferred_element_type=jnp.float32)
    ot_ref[...] = (y + b2t).astype(ot_ref.dtype)


def _round_up(a, m):
    return ((a + m - 1) // m) * m


def kernel(x, w1, b1, w2, b2, *, block_b=65536):
    B, K = x.shape
    inter_dim = w1.shape[1]
    out_dim = w2.shape[1]

    block_b = max(128, min(block_b, _round_up(B, 128)))
    B_pad = _round_up(B, block_b)
    if B_pad != B:
        x = jnp.pad(x, ((0, B_pad - B), (0, 0)))
    num_blocks = B_pad // block_b

    xt = x.T
    w1t = w1.T

    out_t = pl.pallas_call(
        _mlp_t_kernel,
        out_shape=jax.ShapeDtypeStruct((out_dim, B_pad), x.dtype),
        grid=(num_blocks,),
        in_specs=[
            pl.BlockSpec((K, block_b), lambda i: (0, i)),
            pl.BlockSpec((inter_dim, K), lambda i: (0, 0)),
            pl.BlockSpec((1, inter_dim), lambda i: (0, 0)),
            pl.BlockSpec((inter_dim, out_dim), lambda i: (0, 0)),
            pl.BlockSpec((1, out_dim), lambda i: (0, 0)),
        ],
        out_specs=pl.BlockSpec((out_dim, block_b), lambda i: (0, i)),
        compiler_params=pltpu.CompilerParams(
            dimension_semantics=("parallel",),
            vmem_limit_bytes=64 * 1024 * 1024,
        ),
    )(xt, w1t, b1, w2, b2)

    out = out_t.T
    if B_pad != B:
        out = out[:B]
    return out
